# baseline (speedup 1.0000x reference)
import functools

import jax
import jax.numpy as jnp
from jax.experimental import pallas as pl
from jax.experimental.pallas import tpu as pltpu

LANE = 128


def _round_up(x, m):
    return ((x + m - 1) // m) * m


def _enc_kernel(tok_ref, mask_ref, w_ref, b_ref, table_ref, out_ref,
                buf0, buf1, sem0, sem1, *, TM, H, NK):
    k = pl.program_id(1)
    blk = pl.program_id(0) * NK + k
    U = 16

    def issue(base, buf, sem):
        @pl.loop(0, TM // U)
        def _(c):
            for j in range(U):
                g = c * U + j
                pltpu.make_async_copy(
                    table_ref.at[pl.ds(tok_ref[base + g], 1)],
                    buf.at[g], sem.at[j % 4]).start()

    def consume(buf, sem):
        for q in range(4):
            pltpu.make_async_copy(buf.at[pl.ds(0, TM // 4)],
                                  buf.at[pl.ds(0, TM // 4)],
                                  sem.at[q]).wait()
        emb = buf[...].astype(jnp.bfloat16).reshape(TM, H)
        h = jnp.dot(emb, w_ref[...], preferred_element_type=jnp.float32)
        out_ref[...] = jnp.tanh(h + b_ref[...]) * mask_ref[...]

    even = (k % 2) == 0

    @pl.when(k == 0)
    def _():
        issue(blk * TM, buf0, sem0)

    @pl.when(jnp.logical_and(k + 1 < NK, even))
    def _():
        issue((blk + 1) * TM, buf1, sem1)

    @pl.when(jnp.logical_and(k + 1 < NK, jnp.logical_not(even)))
    def _():
        issue((blk + 1) * TM, buf0, sem0)

    @pl.when(even)
    def _():
        consume(buf0, sem0)

    @pl.when(jnp.logical_not(even))
    def _():
        consume(buf1, sem1)


def _encoder_forward(tokens, mask_f32, w_bf16, b_f32, table, *, tm=512):
    M = tokens.shape[0]
    V, H = table.shape
    ncores = 1
    nk = M // tm // ncores
    kernel_body = functools.partial(_enc_kernel, TM=tm, H=H, NK=nk)
    grid_spec = pltpu.PrefetchScalarGridSpec(
        num_scalar_prefetch=1,
        grid=(ncores, nk),
        in_specs=[
            pl.BlockSpec((tm, 1), lambda c, k, tok: (c * nk + k, 0)),
            pl.BlockSpec((H, H), lambda c, k, tok: (0, 0)),
            pl.BlockSpec((1, H), lambda c, k, tok: (0, 0)),
            pl.BlockSpec(memory_space=pl.ANY),
        ],
        out_specs=pl.BlockSpec((tm, H), lambda c, k, tok: (c * nk + k, 0)),
        scratch_shapes=[
            pltpu.VMEM((tm, 1, H), jnp.float32),
            pltpu.VMEM((tm, 1, H), jnp.float32),
            pltpu.SemaphoreType.DMA((4,)),
            pltpu.SemaphoreType.DMA((4,)),
        ],
    )
    return pl.pallas_call(
        kernel_body,
        grid_spec=grid_spec,
        out_shape=jax.ShapeDtypeStruct((M, H), jnp.float32),
        compiler_params=pltpu.CompilerParams(
            dimension_semantics=("parallel", "arbitrary"),
            disable_bounds_checks=True),
        cost_estimate=pl.CostEstimate(
            flops=2 * M * H * H,
            transcendentals=M * H,
            bytes_accessed=(M * H * 4 + M * 4 + H * H * 2 + H * 4
                            + M * H * 4),
        ),
    )(tokens, mask_f32, w_bf16, b_f32, table)


_NT = (((1,), (1,)), ((), ()))


def _head_kernel(span_ref, seq_ref, labels_ref, pack_ref,
                 trig_ref, reps_ref, pwi_ref, cost_ref, ptj_ref,
                 buf_a, buf_b, feat_t, pwi_t, cost_t, ptj_t,
                 sem_a, sem_b, osem,
                 *, n_trig, B, S, N_SPAN, C, H, TG):
    i = pl.program_id(0)
    base = i * TG
    U = 8
    is_trig_step = base + TG <= n_trig

    labels = labels_ref[...]
    l2_row = pack_ref[4:5, 0:C]
    labw2_row = pack_ref[5:6, 0:C]
    b_trig = pack_ref[3:4, 0:1]
    b_type = pack_ref[3:4, 1:2]

    @pl.when(is_trig_step)
    def _():
        @pl.loop(0, TG // U)
        def _(ch):
            for j in range(U):
                g = ch * U + j
                r = base + g
                bt = r // N_SPAN
                st = r % N_SPAN
                off = bt * S
                pltpu.make_async_copy(
                    seq_ref.at[pl.ds(span_ref[bt, st, 0] + off, 1)],
                    buf_a.at[g], sem_a).start()
                pltpu.make_async_copy(
                    seq_ref.at[pl.ds(span_ref[bt, st, 1] + off, 1)],
                    buf_b.at[g], sem_b).start()

        pltpu.make_async_copy(buf_a.at[pl.ds(0, TG)],
                              buf_a.at[pl.ds(0, TG)], sem_a).wait()
        pltpu.make_async_copy(buf_b.at[pl.ds(0, TG)],
                              buf_b.at[pl.ds(0, TG)], sem_b).wait()

        feat = (buf_a[...].reshape(TG, H) + buf_b[...].reshape(TG, H)) * 0.5
        feat_t[...] = feat

        fdots = jax.lax.dot_general(feat, pack_ref[...], _NT,
                                    preferred_element_type=jnp.float32)
        lab_dot = jax.lax.dot_general(feat, labels, _NT,
                                      preferred_element_type=jnp.float32)
        t2 = jnp.sum(feat * feat, axis=-1, keepdims=True)

        pwi_t[...] = jax.nn.sigmoid(fdots[:, 0:1] + b_trig)
        cost_t[...] = jnp.sqrt(jnp.maximum(t2 + l2_row - 2.0 * lab_dot, 0.0))

        c1 = pltpu.make_async_copy(feat_t, trig_ref.at[pl.ds(base, TG)], osem)
        c2 = pltpu.make_async_copy(pwi_t, pwi_ref.at[pl.ds(base, TG)], osem)
        c3 = pltpu.make_async_copy(cost_t, cost_ref.at[pl.ds(base, TG)], osem)
        c1.start(); c2.start(); c3.start()
        c1.wait(); c2.wait(); c3.wait()

    @pl.when(jnp.logical_not(is_trig_step))
    def _():
        @pl.loop(0, B // U)
        def _(ch):
            for j in range(U):
                g = ch * U + j
                pltpu.make_async_copy(seq_ref.at[pl.ds(g * S, 1)],
                                      buf_a.at[g], sem_a).start()

        pltpu.make_async_copy(buf_a.at[pl.ds(0, B)],
                              buf_a.at[pl.ds(0, B)], sem_a).wait()

        feat = buf_a[pl.ds(0, B)].reshape(B, H)
        feat_t[pl.ds(0, B)] = feat

        fdots = jax.lax.dot_general(feat, pack_ref[...], _NT,
                                    preferred_element_type=jnp.float32)
        ptj_t[pl.ds(0, B)] = jax.nn.sigmoid(fdots[:, 1:2] + labw2_row
                                            + b_type)

        c1 = pltpu.make_async_copy(feat_t.at[pl.ds(0, B)], reps_ref, osem)
        c2 = pltpu.make_async_copy(ptj_t.at[pl.ds(0, B)], ptj_ref, osem)
        c1.start(); c2.start()
        c1.wait(); c2.wait()


def _head_forward(span, seq, labels, pack4, *, n_trig, B, S, C, tg):
    M, H = seq.shape
    N_SPAN = span.shape[1]
    GR_pad = n_trig + tg
    kernel_body = functools.partial(
        _head_kernel, n_trig=n_trig, B=B, S=S, N_SPAN=N_SPAN, C=C, H=H, TG=tg)
    grid_spec = pltpu.PrefetchScalarGridSpec(
        num_scalar_prefetch=1,
        grid=(GR_pad // tg,),
        in_specs=[
            pl.BlockSpec(memory_space=pl.ANY),
            pl.BlockSpec((C, H), lambda i, s: (0, 0)),
            pl.BlockSpec((6, H), lambda i, s: (0, 0)),
        ],
        out_specs=(
            pl.BlockSpec(memory_space=pl.ANY),
            pl.BlockSpec(memory_space=pl.ANY),
            pl.BlockSpec(memory_space=pl.ANY),
            pl.BlockSpec(memory_space=pl.ANY),
            pl.BlockSpec(memory_space=pl.ANY),
        ),
        scratch_shapes=[
            pltpu.VMEM((tg, 1, H), jnp.float32),
            pltpu.VMEM((tg, 1, H), jnp.float32),
            pltpu.VMEM((tg, H), jnp.float32),
            pltpu.VMEM((tg, 1), jnp.float32),
            pltpu.VMEM((tg, C), jnp.float32),
            pltpu.VMEM((tg, C), jnp.float32),
            pltpu.SemaphoreType.DMA,
            pltpu.SemaphoreType.DMA,
            pltpu.SemaphoreType.DMA,
        ],
    )
    out_shapes = (
        jax.ShapeDtypeStruct((n_trig, H), jnp.float32),
        jax.ShapeDtypeStruct((B, H), jnp.float32),
        jax.ShapeDtypeStruct((n_trig, 1), jnp.float32),
        jax.ShapeDtypeStruct((n_trig, C), jnp.float32),
        jax.ShapeDtypeStruct((B, C), jnp.float32),
    )
    return pl.pallas_call(
        kernel_body,
        grid_spec=grid_spec,
        out_shape=out_shapes,
        compiler_params=pltpu.CompilerParams(
            dimension_semantics=("parallel",),
            disable_bounds_checks=True,
            vmem_limit_bytes=32 * 1024 * 1024,
        ),
        cost_estimate=pl.CostEstimate(
            flops=2 * GR_pad * H * (C + 4) + 2 * C * C * H,
            transcendentals=2 * GR_pad * C,
            bytes_accessed=(2 * GR_pad * H * 4 + C * H * 4
                            + GR_pad * (H + C + 1) * 4),
        ),
    )(span, seq, labels, pack4)


def kernel(emb_table, w_enc, b_enc, label_embeddings, w_trig, b_trig,
           w_type, b_type, x_tokens, masks, span):
    B, S = x_tokens.shape
    V, H = emb_table.shape
    C = label_embeddings.shape[0]
    N_SPAN = span.shape[1]
    M = B * S
    n_trig = B * N_SPAN

    tokens = jnp.where(masks.reshape(-1) != 0.0,
                       x_tokens.reshape(-1), 0).astype(jnp.int32)
    mask_flat = masks.reshape(-1, 1).astype(jnp.float32)
    seq_f32 = _encoder_forward(tokens, mask_flat,
                               w_enc.astype(jnp.bfloat16), b_enc, emb_table)

    labels = label_embeddings
    bias_row = jnp.pad(jnp.concatenate([b_trig, b_type], axis=1),
                       ((0, 0), (0, H - 2)))
    l2_row = jnp.pad(jnp.sum(labels * labels, axis=1)[None, :],
                     ((0, 0), (0, H - C)))
    labw2_row = jnp.pad((labels @ w_type[H:])[:, 0][None, :],
                        ((0, 0), (0, H - C)))
    pack6 = jnp.concatenate(
        [w_trig.T, w_type.reshape(2, H), bias_row, l2_row, labw2_row],
        axis=0)

    trig_feat, reps, p_wi, cost, ptj2 = _head_forward(
        span.astype(jnp.int32), seq_f32, labels, pack6,
        n_trig=n_trig, B=B, S=S, C=C, tg=128)

    p_tj = ptj2[..., None]
    return {
        "reps": reps,
        "context_feat": seq_f32,
        "trig_feat": trig_feat,
        "p_wi": p_wi,
        "D_W_P": jnp.ones_like(p_wi),
        "p_tj": p_tj,
        "D_T_P": jnp.ones_like(p_tj),
        "cost_matrix": cost,
    }


# speedup vs baseline: 1.3175x; 1.3175x over previous
import functools

import jax
import jax.numpy as jnp
from jax.experimental import pallas as pl
from jax.experimental.pallas import tpu as pltpu

LANE = 128


def _round_up(x, m):
    return ((x + m - 1) // m) * m


def _enc_kernel(tok_ref, mask_ref, w_ref, b_ref, table_ref, out_ref,
                buf0, buf1, zrow, sem0, sem1, *, TM, H, NK):
    k = pl.program_id(1)
    blk = pl.program_id(0) * NK + k
    U = 16

    @pl.when(k == 0)
    def _():
        zrow[...] = jnp.zeros_like(zrow)

    def issue(base, buf, sem):
        @pl.loop(0, TM // U)
        def _(c):
            for j in range(U):
                g = c * U + j
                t = tok_ref[base + g]

                @pl.when(t >= 0)
                def _():
                    pltpu.make_async_copy(
                        table_ref.at[pl.ds(t, 1)],
                        buf.at[g], sem.at[j % 4]).start()

                @pl.when(t < 0)
                def _():
                    pltpu.make_async_copy(
                        zrow, buf.at[g], sem.at[j % 4]).start()

    def consume(buf, sem):
        for q in range(4):
            pltpu.make_async_copy(buf.at[pl.ds(0, TM // 4)],
                                  buf.at[pl.ds(0, TM // 4)],
                                  sem.at[q]).wait()
        emb = buf[...].astype(jnp.bfloat16).reshape(TM, H)
        h = jnp.dot(emb, w_ref[...], preferred_element_type=jnp.float32)
        out_ref[...] = jnp.tanh(h + b_ref[...]) * mask_ref[...]

    even = (k % 2) == 0

    @pl.when(k == 0)
    def _():
        issue(blk * TM, buf0, sem0)

    @pl.when(jnp.logical_and(k + 1 < NK, even))
    def _():
        issue((blk + 1) * TM, buf1, sem1)

    @pl.when(jnp.logical_and(k + 1 < NK, jnp.logical_not(even)))
    def _():
        issue((blk + 1) * TM, buf0, sem0)

    @pl.when(even)
    def _():
        consume(buf0, sem0)

    @pl.when(jnp.logical_not(even))
    def _():
        consume(buf1, sem1)


def _encoder_forward(tokens, mask_f32, w_bf16, b_f32, table, *, tm=512):
    M = tokens.shape[0]
    V, H = table.shape
    ncores = 1
    nk = M // tm // ncores
    kernel_body = functools.partial(_enc_kernel, TM=tm, H=H, NK=nk)
    grid_spec = pltpu.PrefetchScalarGridSpec(
        num_scalar_prefetch=1,
        grid=(ncores, nk),
        in_specs=[
            pl.BlockSpec((tm, 1), lambda c, k, tok: (c * nk + k, 0)),
            pl.BlockSpec((H, H), lambda c, k, tok: (0, 0)),
            pl.BlockSpec((1, H), lambda c, k, tok: (0, 0)),
            pl.BlockSpec(memory_space=pl.ANY),
        ],
        out_specs=pl.BlockSpec((tm, H), lambda c, k, tok: (c * nk + k, 0)),
        scratch_shapes=[
            pltpu.VMEM((tm, 1, H), jnp.float32),
            pltpu.VMEM((tm, 1, H), jnp.float32),
            pltpu.VMEM((1, H), jnp.float32),
            pltpu.SemaphoreType.DMA((4,)),
            pltpu.SemaphoreType.DMA((4,)),
        ],
    )
    return pl.pallas_call(
        kernel_body,
        grid_spec=grid_spec,
        out_shape=jax.ShapeDtypeStruct((M, H), jnp.float32),
        compiler_params=pltpu.CompilerParams(
            dimension_semantics=("parallel", "arbitrary"),
            disable_bounds_checks=True),
        cost_estimate=pl.CostEstimate(
            flops=2 * M * H * H,
            transcendentals=M * H,
            bytes_accessed=(M * H * 4 + M * 4 + H * H * 2 + H * 4
                            + M * H * 4),
        ),
    )(tokens, mask_f32, w_bf16, b_f32, table)


_NT = (((1,), (1,)), ((), ()))


def _head_kernel(span_ref, seq_ref, labels_ref, pack_ref,
                 trig_ref, reps_ref, pwi_ref, cost_ref, ptj_ref,
                 buf_a, buf_b, feat_t, pwi_t, cost_t, ptj_t,
                 sem_a, sem_b, osem,
                 *, n_trig, B, S, N_SPAN, C, H, TG):
    i = pl.program_id(0)
    base = i * TG
    U = 8
    is_trig_step = base + TG <= n_trig

    labels = labels_ref[...]
    l2_row = pack_ref[4:5, 0:C]
    labw2_row = pack_ref[5:6, 0:C]
    b_trig = pack_ref[3:4, 0:1]
    b_type = pack_ref[3:4, 1:2]

    @pl.when(is_trig_step)
    def _():
        @pl.loop(0, TG // U)
        def _(ch):
            for j in range(U):
                g = ch * U + j
                r = base + g
                bt = r // N_SPAN
                st = r % N_SPAN
                off = bt * S
                pltpu.make_async_copy(
                    seq_ref.at[pl.ds(span_ref[bt, st, 0] + off, 1)],
                    buf_a.at[g], sem_a).start()
                pltpu.make_async_copy(
                    seq_ref.at[pl.ds(span_ref[bt, st, 1] + off, 1)],
                    buf_b.at[g], sem_b).start()

        pltpu.make_async_copy(buf_a.at[pl.ds(0, TG)],
                              buf_a.at[pl.ds(0, TG)], sem_a).wait()
        pltpu.make_async_copy(buf_b.at[pl.ds(0, TG)],
                              buf_b.at[pl.ds(0, TG)], sem_b).wait()

        feat = (buf_a[...].reshape(TG, H) + buf_b[...].reshape(TG, H)) * 0.5
        feat_t[...] = feat

        fdots = jax.lax.dot_general(feat, pack_ref[...], _NT,
                                    preferred_element_type=jnp.float32)
        lab_dot = jax.lax.dot_general(feat, labels, _NT,
                                      preferred_element_type=jnp.float32)
        t2 = jnp.sum(feat * feat, axis=-1, keepdims=True)

        pwi_t[...] = jax.nn.sigmoid(fdots[:, 0:1] + b_trig)
        cost_t[...] = jnp.sqrt(jnp.maximum(t2 + l2_row - 2.0 * lab_dot, 0.0))

        c1 = pltpu.make_async_copy(feat_t, trig_ref.at[pl.ds(base, TG)], osem)
        c2 = pltpu.make_async_copy(pwi_t, pwi_ref.at[pl.ds(base, TG)], osem)
        c3 = pltpu.make_async_copy(cost_t, cost_ref.at[pl.ds(base, TG)], osem)
        c1.start(); c2.start(); c3.start()
        c1.wait(); c2.wait(); c3.wait()

    @pl.when(jnp.logical_not(is_trig_step))
    def _():
        @pl.loop(0, B // U)
        def _(ch):
            for j in range(U):
                g = ch * U + j
                pltpu.make_async_copy(seq_ref.at[pl.ds(g * S, 1)],
                                      buf_a.at[g], sem_a).start()

        pltpu.make_async_copy(buf_a.at[pl.ds(0, B)],
                              buf_a.at[pl.ds(0, B)], sem_a).wait()

        feat = buf_a[pl.ds(0, B)].reshape(B, H)
        feat_t[pl.ds(0, B)] = feat

        fdots = jax.lax.dot_general(feat, pack_ref[...], _NT,
                                    preferred_element_type=jnp.float32)
        ptj_t[pl.ds(0, B)] = jax.nn.sigmoid(fdots[:, 1:2] + labw2_row
                                            + b_type)

        c1 = pltpu.make_async_copy(feat_t.at[pl.ds(0, B)], reps_ref, osem)
        c2 = pltpu.make_async_copy(ptj_t.at[pl.ds(0, B)], ptj_ref, osem)
        c1.start(); c2.start()
        c1.wait(); c2.wait()


def _head_forward(span, seq, labels, pack4, *, n_trig, B, S, C, tg):
    M, H = seq.shape
    N_SPAN = span.shape[1]
    GR_pad = n_trig + tg
    kernel_body = functools.partial(
        _head_kernel, n_trig=n_trig, B=B, S=S, N_SPAN=N_SPAN, C=C, H=H, TG=tg)
    grid_spec = pltpu.PrefetchScalarGridSpec(
        num_scalar_prefetch=1,
        grid=(GR_pad // tg,),
        in_specs=[
            pl.BlockSpec(memory_space=pl.ANY),
            pl.BlockSpec((C, H), lambda i, s: (0, 0)),
            pl.BlockSpec((6, H), lambda i, s: (0, 0)),
        ],
        out_specs=(
            pl.BlockSpec(memory_space=pl.ANY),
            pl.BlockSpec(memory_space=pl.ANY),
            pl.BlockSpec(memory_space=pl.ANY),
            pl.BlockSpec(memory_space=pl.ANY),
            pl.BlockSpec(memory_space=pl.ANY),
        ),
        scratch_shapes=[
            pltpu.VMEM((tg, 1, H), jnp.float32),
            pltpu.VMEM((tg, 1, H), jnp.float32),
            pltpu.VMEM((tg, H), jnp.float32),
            pltpu.VMEM((tg, 1), jnp.float32),
            pltpu.VMEM((tg, C), jnp.float32),
            pltpu.VMEM((tg, C), jnp.float32),
            pltpu.SemaphoreType.DMA,
            pltpu.SemaphoreType.DMA,
            pltpu.SemaphoreType.DMA,
        ],
    )
    out_shapes = (
        jax.ShapeDtypeStruct((n_trig, H), jnp.float32),
        jax.ShapeDtypeStruct((B, H), jnp.float32),
        jax.ShapeDtypeStruct((n_trig, 1), jnp.float32),
        jax.ShapeDtypeStruct((n_trig, C), jnp.float32),
        jax.ShapeDtypeStruct((B, C), jnp.float32),
    )
    return pl.pallas_call(
        kernel_body,
        grid_spec=grid_spec,
        out_shape=out_shapes,
        compiler_params=pltpu.CompilerParams(
            dimension_semantics=("parallel",),
            disable_bounds_checks=True,
            vmem_limit_bytes=32 * 1024 * 1024,
        ),
        cost_estimate=pl.CostEstimate(
            flops=2 * GR_pad * H * (C + 4) + 2 * C * C * H,
            transcendentals=2 * GR_pad * C,
            bytes_accessed=(2 * GR_pad * H * 4 + C * H * 4
                            + GR_pad * (H + C + 1) * 4),
        ),
    )(span, seq, labels, pack4)


def kernel(emb_table, w_enc, b_enc, label_embeddings, w_trig, b_trig,
           w_type, b_type, x_tokens, masks, span):
    B, S = x_tokens.shape
    V, H = emb_table.shape
    C = label_embeddings.shape[0]
    N_SPAN = span.shape[1]
    M = B * S
    n_trig = B * N_SPAN

    tokens = jnp.where(masks.reshape(-1) != 0.0,
                       x_tokens.reshape(-1), -1).astype(jnp.int32)
    mask_flat = masks.reshape(-1, 1).astype(jnp.float32)
    seq_f32 = _encoder_forward(tokens, mask_flat,
                               w_enc.astype(jnp.bfloat16), b_enc, emb_table)

    labels = label_embeddings
    bias_row = jnp.pad(jnp.concatenate([b_trig, b_type], axis=1),
                       ((0, 0), (0, H - 2)))
    l2_row = jnp.pad(jnp.sum(labels * labels, axis=1)[None, :],
                     ((0, 0), (0, H - C)))
    labw2_row = jnp.pad((labels @ w_type[H:])[:, 0][None, :],
                        ((0, 0), (0, H - C)))
    pack6 = jnp.concatenate(
        [w_trig.T, w_type.reshape(2, H), bias_row, l2_row, labw2_row],
        axis=0)

    trig_feat, reps, p_wi, cost, ptj2 = _head_forward(
        span.astype(jnp.int32), seq_f32, labels, pack6,
        n_trig=n_trig, B=B, S=S, C=C, tg=128)

    p_tj = ptj2[..., None]
    return {
        "reps": reps,
        "context_feat": seq_f32,
        "trig_feat": trig_feat,
        "p_wi": p_wi,
        "D_W_P": jnp.ones_like(p_wi),
        "p_tj": p_tj,
        "D_T_P": jnp.ones_like(p_tj),
        "cost_matrix": cost,
    }


# speedup vs baseline: 1.7536x; 1.3310x over previous
import functools

import jax
import jax.numpy as jnp
from jax.experimental import pallas as pl
from jax.experimental.pallas import tpu as pltpu

LANE = 128


def _round_up(x, m):
    return ((x + m - 1) // m) * m


def _enc_kernel(tok_ref, mask_ref, w_ref, b_ref, table_ref, out_ref,
                buf0, buf1, sem0, sem1, *, TM, H, NK):
    k = pl.program_id(1)
    blk = pl.program_id(0) * NK + k
    U = 16

    def issue(base, buf, sem):
        @pl.loop(0, TM // U)
        def _(c):
            for j in range(U):
                g = c * U + j
                pltpu.make_async_copy(
                    table_ref.at[pl.ds(tok_ref[base + g], 1)],
                    buf.at[g], sem.at[j % 4]).start()

    def consume(buf, sem):
        for q in range(4):
            pltpu.make_async_copy(buf.at[pl.ds(0, TM // 4)],
                                  buf.at[pl.ds(0, TM // 4)],
                                  sem.at[q]).wait()
        emb = buf[...].astype(jnp.bfloat16).reshape(TM, H)
        h = jnp.dot(emb, w_ref[...], preferred_element_type=jnp.float32)
        out_ref[...] = jnp.tanh(h + b_ref[...]) * mask_ref[...]

    even = (k % 2) == 0

    @pl.when(k == 0)
    def _():
        issue(blk * TM, buf0, sem0)

    @pl.when(jnp.logical_and(k + 1 < NK, even))
    def _():
        issue((blk + 1) * TM, buf1, sem1)

    @pl.when(jnp.logical_and(k + 1 < NK, jnp.logical_not(even)))
    def _():
        issue((blk + 1) * TM, buf0, sem0)

    @pl.when(even)
    def _():
        consume(buf0, sem0)

    @pl.when(jnp.logical_not(even))
    def _():
        consume(buf1, sem1)


def _encoder_forward(tokens, mask_f32, w_bf16, b_f32, table, *, tm=512):
    M = tokens.shape[0]
    V, H = table.shape
    ncores = 1
    nk = M // tm // ncores
    kernel_body = functools.partial(_enc_kernel, TM=tm, H=H, NK=nk)
    grid_spec = pltpu.PrefetchScalarGridSpec(
        num_scalar_prefetch=1,
        grid=(ncores, nk),
        in_specs=[
            pl.BlockSpec((tm, 1), lambda c, k, tok: (c * nk + k, 0)),
            pl.BlockSpec((H, H), lambda c, k, tok: (0, 0)),
            pl.BlockSpec((1, H), lambda c, k, tok: (0, 0)),
            pl.BlockSpec(memory_space=pl.ANY),
        ],
        out_specs=pl.BlockSpec((tm, H), lambda c, k, tok: (c * nk + k, 0)),
        scratch_shapes=[
            pltpu.VMEM((tm, 1, H), jnp.float32),
            pltpu.VMEM((tm, 1, H), jnp.float32),
            pltpu.SemaphoreType.DMA((4,)),
            pltpu.SemaphoreType.DMA((4,)),
        ],
    )
    return pl.pallas_call(
        kernel_body,
        grid_spec=grid_spec,
        out_shape=jax.ShapeDtypeStruct((M, H), jnp.float32),
        compiler_params=pltpu.CompilerParams(
            dimension_semantics=("parallel", "arbitrary"),
            disable_bounds_checks=True),
        cost_estimate=pl.CostEstimate(
            flops=2 * M * H * H,
            transcendentals=M * H,
            bytes_accessed=(M * H * 4 + M * 4 + H * H * 2 + H * 4
                            + M * H * 4),
        ),
    )(tokens, mask_f32, w_bf16, b_f32, table)


_NT = (((1,), (1,)), ((), ()))


def _head_kernel(span_ref, seq_ref, labels_ref, pack_ref,
                 trig_ref, reps_ref, pwi_ref, cost_ref, ptj_ref,
                 buf_a0, buf_b0, buf_a1, buf_b1,
                 feat_t, pwi_t, cost_t, ptj_t,
                 sem_a0, sem_b0, sem_a1, sem_b1, osem,
                 *, n_trig, B, S, N_SPAN, C, H, TG, NSTEPS):
    k = pl.program_id(0)
    U = 8

    def issue(blk, buf_a, buf_b, sem_a, sem_b):
        base = blk * TG
        trig_blk = base + TG <= n_trig

        @pl.when(trig_blk)
        def _():
            @pl.loop(0, TG // U)
            def _(ch):
                for j in range(U):
                    g = ch * U + j
                    r = base + g
                    bt = r // N_SPAN
                    st = r % N_SPAN
                    off = bt * S
                    pltpu.make_async_copy(
                        seq_ref.at[pl.ds(span_ref[bt, st, 0] + off, 1)],
                        buf_a.at[g], sem_a).start()
                    pltpu.make_async_copy(
                        seq_ref.at[pl.ds(span_ref[bt, st, 1] + off, 1)],
                        buf_b.at[g], sem_b).start()

        @pl.when(jnp.logical_not(trig_blk))
        def _():
            @pl.loop(0, B // U)
            def _(ch):
                for j in range(U):
                    g = ch * U + j
                    pltpu.make_async_copy(seq_ref.at[pl.ds(g * S, 1)],
                                          buf_a.at[g], sem_a).start()

    def consume(blk, buf_a, buf_b, sem_a, sem_b):
        base = blk * TG
        trig_blk = base + TG <= n_trig
        labels = labels_ref[...]
        l2_row = pack_ref[4:5, 0:C]
        labw2_row = pack_ref[5:6, 0:C]
        b_trig = pack_ref[3:4, 0:1]
        b_type = pack_ref[3:4, 1:2]

        @pl.when(trig_blk)
        def _():
            pltpu.make_async_copy(buf_a.at[pl.ds(0, TG)],
                                  buf_a.at[pl.ds(0, TG)], sem_a).wait()
            pltpu.make_async_copy(buf_b.at[pl.ds(0, TG)],
                                  buf_b.at[pl.ds(0, TG)], sem_b).wait()

            feat = (buf_a[...].reshape(TG, H)
                    + buf_b[...].reshape(TG, H)) * 0.5
            feat_t[...] = feat

            fdots = jax.lax.dot_general(feat, pack_ref[...], _NT,
                                        preferred_element_type=jnp.float32)
            lab_dot = jax.lax.dot_general(feat, labels, _NT,
                                          preferred_element_type=jnp.float32)
            t2 = jnp.sum(feat * feat, axis=-1, keepdims=True)

            pwi_t[...] = jax.nn.sigmoid(fdots[:, 0:1] + b_trig)
            cost_t[...] = jnp.sqrt(jnp.maximum(t2 + l2_row - 2.0 * lab_dot,
                                               0.0))

            c1 = pltpu.make_async_copy(feat_t, trig_ref.at[pl.ds(base, TG)],
                                       osem)
            c2 = pltpu.make_async_copy(pwi_t, pwi_ref.at[pl.ds(base, TG)],
                                       osem)
            c3 = pltpu.make_async_copy(cost_t, cost_ref.at[pl.ds(base, TG)],
                                       osem)
            c1.start(); c2.start(); c3.start()
            c1.wait(); c2.wait(); c3.wait()

        @pl.when(jnp.logical_not(trig_blk))
        def _():
            pltpu.make_async_copy(buf_a.at[pl.ds(0, B)],
                                  buf_a.at[pl.ds(0, B)], sem_a).wait()

            feat = buf_a[pl.ds(0, B)].reshape(B, H)
            feat_t[pl.ds(0, B)] = feat

            fdots = jax.lax.dot_general(feat, pack_ref[...], _NT,
                                        preferred_element_type=jnp.float32)
            ptj_t[pl.ds(0, B)] = jax.nn.sigmoid(fdots[:, 1:2] + labw2_row
                                                + b_type)

            c1 = pltpu.make_async_copy(feat_t.at[pl.ds(0, B)], reps_ref,
                                       osem)
            c2 = pltpu.make_async_copy(ptj_t.at[pl.ds(0, B)], ptj_ref, osem)
            c1.start(); c2.start()
            c1.wait(); c2.wait()

    even = (k % 2) == 0
    have_next = k + 1 < NSTEPS

    @pl.when(k == 0)
    def _():
        issue(k, buf_a0, buf_b0, sem_a0, sem_b0)

    @pl.when(jnp.logical_and(have_next, even))
    def _():
        issue(k + 1, buf_a1, buf_b1, sem_a1, sem_b1)

    @pl.when(jnp.logical_and(have_next, jnp.logical_not(even)))
    def _():
        issue(k + 1, buf_a0, buf_b0, sem_a0, sem_b0)

    @pl.when(even)
    def _():
        consume(k, buf_a0, buf_b0, sem_a0, sem_b0)

    @pl.when(jnp.logical_not(even))
    def _():
        consume(k, buf_a1, buf_b1, sem_a1, sem_b1)


def _head_forward(span, seq, labels, pack4, *, n_trig, B, S, C, tg):
    M, H = seq.shape
    N_SPAN = span.shape[1]
    GR_pad = n_trig + tg
    nsteps = GR_pad // tg
    kernel_body = functools.partial(
        _head_kernel, n_trig=n_trig, B=B, S=S, N_SPAN=N_SPAN, C=C, H=H,
        TG=tg, NSTEPS=nsteps)
    grid_spec = pltpu.PrefetchScalarGridSpec(
        num_scalar_prefetch=1,
        grid=(GR_pad // tg,),
        in_specs=[
            pl.BlockSpec(memory_space=pl.ANY),
            pl.BlockSpec((C, H), lambda i, s: (0, 0)),
            pl.BlockSpec((6, H), lambda i, s: (0, 0)),
        ],
        out_specs=(
            pl.BlockSpec(memory_space=pl.ANY),
            pl.BlockSpec(memory_space=pl.ANY),
            pl.BlockSpec(memory_space=pl.ANY),
            pl.BlockSpec(memory_space=pl.ANY),
            pl.BlockSpec(memory_space=pl.ANY),
        ),
        scratch_shapes=[
            pltpu.VMEM((tg, 1, H), jnp.float32),
            pltpu.VMEM((tg, 1, H), jnp.float32),
            pltpu.VMEM((tg, 1, H), jnp.float32),
            pltpu.VMEM((tg, 1, H), jnp.float32),
            pltpu.VMEM((tg, H), jnp.float32),
            pltpu.VMEM((tg, 1), jnp.float32),
            pltpu.VMEM((tg, C), jnp.float32),
            pltpu.VMEM((tg, C), jnp.float32),
            pltpu.SemaphoreType.DMA,
            pltpu.SemaphoreType.DMA,
            pltpu.SemaphoreType.DMA,
            pltpu.SemaphoreType.DMA,
            pltpu.SemaphoreType.DMA,
        ],
    )
    out_shapes = (
        jax.ShapeDtypeStruct((n_trig, H), jnp.float32),
        jax.ShapeDtypeStruct((B, H), jnp.float32),
        jax.ShapeDtypeStruct((n_trig, 1), jnp.float32),
        jax.ShapeDtypeStruct((n_trig, C), jnp.float32),
        jax.ShapeDtypeStruct((B, C), jnp.float32),
    )
    return pl.pallas_call(
        kernel_body,
        grid_spec=grid_spec,
        out_shape=out_shapes,
        compiler_params=pltpu.CompilerParams(
            dimension_semantics=("arbitrary",),
            disable_bounds_checks=True,
            vmem_limit_bytes=32 * 1024 * 1024,
        ),
        cost_estimate=pl.CostEstimate(
            flops=2 * GR_pad * H * (C + 4) + 2 * C * C * H,
            transcendentals=2 * GR_pad * C,
            bytes_accessed=(2 * GR_pad * H * 4 + C * H * 4
                            + GR_pad * (H + C + 1) * 4),
        ),
    )(span, seq, labels, pack4)


def kernel(emb_table, w_enc, b_enc, label_embeddings, w_trig, b_trig,
           w_type, b_type, x_tokens, masks, span):
    B, S = x_tokens.shape
    V, H = emb_table.shape
    C = label_embeddings.shape[0]
    N_SPAN = span.shape[1]
    M = B * S
    n_trig = B * N_SPAN

    tokens = x_tokens.reshape(-1).astype(jnp.int32)
    mask_flat = masks.reshape(-1, 1).astype(jnp.float32)
    seq_f32 = _encoder_forward(tokens, mask_flat,
                               w_enc.astype(jnp.bfloat16), b_enc, emb_table)

    labels = label_embeddings
    bias_row = jnp.pad(jnp.concatenate([b_trig, b_type], axis=1),
                       ((0, 0), (0, H - 2)))
    l2_row = jnp.pad(jnp.sum(labels * labels, axis=1)[None, :],
                     ((0, 0), (0, H - C)))
    labw2_row = jnp.pad((labels @ w_type[H:])[:, 0][None, :],
                        ((0, 0), (0, H - C)))
    pack6 = jnp.concatenate(
        [w_trig.T, w_type.reshape(2, H), bias_row, l2_row, labw2_row],
        axis=0)

    trig_feat, reps, p_wi, cost, ptj2 = _head_forward(
        span.astype(jnp.int32), seq_f32, labels, pack6,
        n_trig=n_trig, B=B, S=S, C=C, tg=128)

    p_tj = ptj2[..., None]
    return {
        "reps": reps,
        "context_feat": seq_f32,
        "trig_feat": trig_feat,
        "p_wi": p_wi,
        "D_W_P": jnp.ones_like(p_wi),
        "p_tj": p_tj,
        "D_T_P": jnp.ones_like(p_tj),
        "cost_matrix": cost,
    }


# speedup vs baseline: 1.7972x; 1.0249x over previous
import functools

import jax
import jax.numpy as jnp
from jax.experimental import pallas as pl
from jax.experimental.pallas import tpu as pltpu

LANE = 128


def _round_up(x, m):
    return ((x + m - 1) // m) * m


def _enc_kernel(tok_ref, mask_ref, w_ref, b_ref, table_ref, out_ref,
                buf0, buf1, sem0, sem1, *, TM, H, NK):
    k = pl.program_id(1)
    blk = pl.program_id(0) * NK + k
    U = 16

    def issue(base, buf, sem):
        for g in range(TM):
            pltpu.make_async_copy(
                table_ref.at[pl.ds(tok_ref[base + g], 1)],
                buf.at[g], sem.at[g % 4]).start()

    def consume(buf, sem):
        for q in range(4):
            pltpu.make_async_copy(buf.at[pl.ds(0, TM // 4)],
                                  buf.at[pl.ds(0, TM // 4)],
                                  sem.at[q]).wait()
        emb = buf[...].astype(jnp.bfloat16).reshape(TM, H)
        h = jnp.dot(emb, w_ref[...], preferred_element_type=jnp.float32)
        out_ref[...] = jnp.tanh(h + b_ref[...]) * mask_ref[...]

    even = (k % 2) == 0

    @pl.when(k == 0)
    def _():
        issue(blk * TM, buf0, sem0)

    @pl.when(even)
    def _():
        issue((blk + 1) * TM, buf1, sem1)
        consume(buf0, sem0)

    @pl.when(jnp.logical_and(jnp.logical_not(even), k < NK - 1))
    def _():
        issue((blk + 1) * TM, buf0, sem0)
        consume(buf1, sem1)

    @pl.when(k == NK - 1)
    def _():
        consume(buf1, sem1)


def _encoder_forward(tokens, mask_f32, w_bf16, b_f32, table, *, tm=512):
    M = tokens.shape[0]
    V, H = table.shape
    ncores = 1
    nk = M // tm // ncores
    kernel_body = functools.partial(_enc_kernel, TM=tm, H=H, NK=nk)
    grid_spec = pltpu.PrefetchScalarGridSpec(
        num_scalar_prefetch=1,
        grid=(ncores, nk),
        in_specs=[
            pl.BlockSpec((tm, 1), lambda c, k, tok: (c * nk + k, 0)),
            pl.BlockSpec((H, H), lambda c, k, tok: (0, 0)),
            pl.BlockSpec((1, H), lambda c, k, tok: (0, 0)),
            pl.BlockSpec(memory_space=pl.ANY),
        ],
        out_specs=pl.BlockSpec((tm, H), lambda c, k, tok: (c * nk + k, 0)),
        scratch_shapes=[
            pltpu.VMEM((tm, 1, H), jnp.float32),
            pltpu.VMEM((tm, 1, H), jnp.float32),
            pltpu.SemaphoreType.DMA((4,)),
            pltpu.SemaphoreType.DMA((4,)),
        ],
    )
    return pl.pallas_call(
        kernel_body,
        grid_spec=grid_spec,
        out_shape=jax.ShapeDtypeStruct((M, H), jnp.float32),
        compiler_params=pltpu.CompilerParams(
            dimension_semantics=("parallel", "arbitrary"),
            disable_bounds_checks=True),
        cost_estimate=pl.CostEstimate(
            flops=2 * M * H * H,
            transcendentals=M * H,
            bytes_accessed=(M * H * 4 + M * 4 + H * H * 2 + H * 4
                            + M * H * 4),
        ),
    )(tokens, mask_f32, w_bf16, b_f32, table)


_NT = (((1,), (1,)), ((), ()))


def _head_kernel(span_ref, seq_ref, labels_ref, pack_ref,
                 trig_ref, reps_ref, pwi_ref, cost_ref, ptj_ref,
                 buf_a0, buf_b0, buf_a1, buf_b1,
                 feat_t, pwi_t, cost_t, ptj_t,
                 sem_a0, sem_b0, sem_a1, sem_b1, osem,
                 *, n_trig, B, S, N_SPAN, C, H, TG, NSTEPS):
    k = pl.program_id(0)
    U = 8

    def issue(blk, buf_a, buf_b, sem_a, sem_b):
        base = blk * TG
        trig_blk = base + TG <= n_trig

        @pl.when(trig_blk)
        def _():
            @pl.loop(0, TG // U)
            def _(ch):
                for j in range(U):
                    g = ch * U + j
                    r = base + g
                    bt = r // N_SPAN
                    st = r % N_SPAN
                    off = bt * S
                    pltpu.make_async_copy(
                        seq_ref.at[pl.ds(span_ref[bt, st, 0] + off, 1)],
                        buf_a.at[g], sem_a).start()
                    pltpu.make_async_copy(
                        seq_ref.at[pl.ds(span_ref[bt, st, 1] + off, 1)],
                        buf_b.at[g], sem_b).start()

        @pl.when(jnp.logical_not(trig_blk))
        def _():
            @pl.loop(0, B // U)
            def _(ch):
                for j in range(U):
                    g = ch * U + j
                    pltpu.make_async_copy(seq_ref.at[pl.ds(g * S, 1)],
                                          buf_a.at[g], sem_a).start()

    def consume(blk, buf_a, buf_b, sem_a, sem_b):
        base = blk * TG
        trig_blk = base + TG <= n_trig
        labels = labels_ref[...]
        l2_row = pack_ref[4:5, 0:C]
        labw2_row = pack_ref[5:6, 0:C]
        b_trig = pack_ref[3:4, 0:1]
        b_type = pack_ref[3:4, 1:2]

        @pl.when(trig_blk)
        def _():
            pltpu.make_async_copy(buf_a.at[pl.ds(0, TG)],
                                  buf_a.at[pl.ds(0, TG)], sem_a).wait()
            pltpu.make_async_copy(buf_b.at[pl.ds(0, TG)],
                                  buf_b.at[pl.ds(0, TG)], sem_b).wait()

            feat = (buf_a[...].reshape(TG, H)
                    + buf_b[...].reshape(TG, H)) * 0.5
            feat_t[...] = feat

            fdots = jax.lax.dot_general(feat, pack_ref[...], _NT,
                                        preferred_element_type=jnp.float32)
            lab_dot = jax.lax.dot_general(feat, labels, _NT,
                                          preferred_element_type=jnp.float32)
            t2 = jnp.sum(feat * feat, axis=-1, keepdims=True)

            pwi_t[...] = jax.nn.sigmoid(fdots[:, 0:1] + b_trig)
            cost_t[...] = jnp.sqrt(jnp.maximum(t2 + l2_row - 2.0 * lab_dot,
                                               0.0))

            c1 = pltpu.make_async_copy(feat_t, trig_ref.at[pl.ds(base, TG)],
                                       osem)
            c2 = pltpu.make_async_copy(pwi_t, pwi_ref.at[pl.ds(base, TG)],
                                       osem)
            c3 = pltpu.make_async_copy(cost_t, cost_ref.at[pl.ds(base, TG)],
                                       osem)
            c1.start(); c2.start(); c3.start()
            c1.wait(); c2.wait(); c3.wait()

        @pl.when(jnp.logical_not(trig_blk))
        def _():
            pltpu.make_async_copy(buf_a.at[pl.ds(0, B)],
                                  buf_a.at[pl.ds(0, B)], sem_a).wait()

            feat = buf_a[pl.ds(0, B)].reshape(B, H)
            feat_t[pl.ds(0, B)] = feat

            fdots = jax.lax.dot_general(feat, pack_ref[...], _NT,
                                        preferred_element_type=jnp.float32)
            ptj_t[pl.ds(0, B)] = jax.nn.sigmoid(fdots[:, 1:2] + labw2_row
                                                + b_type)

            c1 = pltpu.make_async_copy(feat_t.at[pl.ds(0, B)], reps_ref,
                                       osem)
            c2 = pltpu.make_async_copy(ptj_t.at[pl.ds(0, B)], ptj_ref, osem)
            c1.start(); c2.start()
            c1.wait(); c2.wait()

    even = (k % 2) == 0
    have_next = k + 1 < NSTEPS

    @pl.when(k == 0)
    def _():
        issue(k, buf_a0, buf_b0, sem_a0, sem_b0)

    @pl.when(jnp.logical_and(have_next, even))
    def _():
        issue(k + 1, buf_a1, buf_b1, sem_a1, sem_b1)

    @pl.when(jnp.logical_and(have_next, jnp.logical_not(even)))
    def _():
        issue(k + 1, buf_a0, buf_b0, sem_a0, sem_b0)

    @pl.when(even)
    def _():
        consume(k, buf_a0, buf_b0, sem_a0, sem_b0)

    @pl.when(jnp.logical_not(even))
    def _():
        consume(k, buf_a1, buf_b1, sem_a1, sem_b1)


def _head_forward(span, seq, labels, pack4, *, n_trig, B, S, C, tg):
    M, H = seq.shape
    N_SPAN = span.shape[1]
    GR_pad = n_trig + tg
    nsteps = GR_pad // tg
    kernel_body = functools.partial(
        _head_kernel, n_trig=n_trig, B=B, S=S, N_SPAN=N_SPAN, C=C, H=H,
        TG=tg, NSTEPS=nsteps)
    grid_spec = pltpu.PrefetchScalarGridSpec(
        num_scalar_prefetch=1,
        grid=(GR_pad // tg,),
        in_specs=[
            pl.BlockSpec(memory_space=pl.ANY),
            pl.BlockSpec((C, H), lambda i, s: (0, 0)),
            pl.BlockSpec((6, H), lambda i, s: (0, 0)),
        ],
        out_specs=(
            pl.BlockSpec(memory_space=pl.ANY),
            pl.BlockSpec(memory_space=pl.ANY),
            pl.BlockSpec(memory_space=pl.ANY),
            pl.BlockSpec(memory_space=pl.ANY),
            pl.BlockSpec(memory_space=pl.ANY),
        ),
        scratch_shapes=[
            pltpu.VMEM((tg, 1, H), jnp.float32),
            pltpu.VMEM((tg, 1, H), jnp.float32),
            pltpu.VMEM((tg, 1, H), jnp.float32),
            pltpu.VMEM((tg, 1, H), jnp.float32),
            pltpu.VMEM((tg, H), jnp.float32),
            pltpu.VMEM((tg, 1), jnp.float32),
            pltpu.VMEM((tg, C), jnp.float32),
            pltpu.VMEM((tg, C), jnp.float32),
            pltpu.SemaphoreType.DMA,
            pltpu.SemaphoreType.DMA,
            pltpu.SemaphoreType.DMA,
            pltpu.SemaphoreType.DMA,
            pltpu.SemaphoreType.DMA,
        ],
    )
    out_shapes = (
        jax.ShapeDtypeStruct((n_trig, H), jnp.float32),
        jax.ShapeDtypeStruct((B, H), jnp.float32),
        jax.ShapeDtypeStruct((n_trig, 1), jnp.float32),
        jax.ShapeDtypeStruct((n_trig, C), jnp.float32),
        jax.ShapeDtypeStruct((B, C), jnp.float32),
    )
    return pl.pallas_call(
        kernel_body,
        grid_spec=grid_spec,
        out_shape=out_shapes,
        compiler_params=pltpu.CompilerParams(
            dimension_semantics=("arbitrary",),
            disable_bounds_checks=True,
            vmem_limit_bytes=32 * 1024 * 1024,
        ),
        cost_estimate=pl.CostEstimate(
            flops=2 * GR_pad * H * (C + 4) + 2 * C * C * H,
            transcendentals=2 * GR_pad * C,
            bytes_accessed=(2 * GR_pad * H * 4 + C * H * 4
                            + GR_pad * (H + C + 1) * 4),
        ),
    )(span, seq, labels, pack4)


def kernel(emb_table, w_enc, b_enc, label_embeddings, w_trig, b_trig,
           w_type, b_type, x_tokens, masks, span):
    B, S = x_tokens.shape
    V, H = emb_table.shape
    C = label_embeddings.shape[0]
    N_SPAN = span.shape[1]
    M = B * S
    n_trig = B * N_SPAN

    tokens = x_tokens.reshape(-1).astype(jnp.int32)
    mask_flat = masks.reshape(-1, 1).astype(jnp.float32)
    seq_f32 = _encoder_forward(tokens, mask_flat,
                               w_enc.astype(jnp.bfloat16), b_enc, emb_table)

    labels = label_embeddings
    bias_row = jnp.pad(jnp.concatenate([b_trig, b_type], axis=1),
                       ((0, 0), (0, H - 2)))
    l2_row = jnp.pad(jnp.sum(labels * labels, axis=1)[None, :],
                     ((0, 0), (0, H - C)))
    labw2_row = jnp.pad((labels @ w_type[H:])[:, 0][None, :],
                        ((0, 0), (0, H - C)))
    pack6 = jnp.concatenate(
        [w_trig.T, w_type.reshape(2, H), bias_row, l2_row, labw2_row],
        axis=0)

    trig_feat, reps, p_wi, cost, ptj2 = _head_forward(
        span.astype(jnp.int32), seq_f32, labels, pack6,
        n_trig=n_trig, B=B, S=S, C=C, tg=128)

    p_tj = ptj2[..., None]
    return {
        "reps": reps,
        "context_feat": seq_f32,
        "trig_feat": trig_feat,
        "p_wi": p_wi,
        "D_W_P": jnp.ones_like(p_wi),
        "p_tj": p_tj,
        "D_T_P": jnp.ones_like(p_tj),
        "cost_matrix": cost,
    }


# speedup vs baseline: 1.8635x; 1.0369x over previous
import functools

import jax
import jax.numpy as jnp
from jax.experimental import pallas as pl
from jax.experimental.pallas import tpu as pltpu

LANE = 128


def _round_up(x, m):
    return ((x + m - 1) // m) * m


def _enc_kernel(tok_ref, mask_ref, w_ref, b_ref, table_ref, out_ref,
                buf0, buf1, sem0, sem1, *, TM, H, NK):
    k = pl.program_id(1)
    blk = pl.program_id(0) * NK + k
    U = 16

    def issue(base, buf, sem):
        for g in range(TM):
            pltpu.make_async_copy(
                table_ref.at[pl.ds(tok_ref[base + g], 1)],
                buf.at[g], sem.at[g % 4]).start()

    def consume(buf, sem):
        for q in range(4):
            pltpu.make_async_copy(buf.at[pl.ds(0, TM // 4)],
                                  buf.at[pl.ds(0, TM // 4)],
                                  sem.at[q]).wait()
        emb = buf[...].astype(jnp.bfloat16).reshape(TM, H)
        h = jnp.dot(emb, w_ref[...].astype(jnp.bfloat16),
                    preferred_element_type=jnp.float32)
        out_ref[...] = jnp.tanh(h + b_ref[...]) * mask_ref[...]

    even = (k % 2) == 0

    @pl.when(k == 0)
    def _():
        issue(blk * TM, buf0, sem0)

    @pl.when(even)
    def _():
        issue((blk + 1) * TM, buf1, sem1)
        consume(buf0, sem0)

    @pl.when(jnp.logical_and(jnp.logical_not(even), k < NK - 1))
    def _():
        issue((blk + 1) * TM, buf0, sem0)
        consume(buf1, sem1)

    @pl.when(k == NK - 1)
    def _():
        consume(buf1, sem1)


def _encoder_forward(tokens, mask_f32, w_bf16, b_f32, table, *, tm=512):
    M = tokens.shape[0]
    V, H = table.shape
    ncores = 1
    nk = M // tm // ncores
    kernel_body = functools.partial(_enc_kernel, TM=tm, H=H, NK=nk)
    grid_spec = pltpu.PrefetchScalarGridSpec(
        num_scalar_prefetch=1,
        grid=(ncores, nk),
        in_specs=[
            pl.BlockSpec((tm, 1), lambda c, k, tok: (c * nk + k, 0)),
            pl.BlockSpec((H, H), lambda c, k, tok: (0, 0)),
            pl.BlockSpec((1, H), lambda c, k, tok: (0, 0)),
            pl.BlockSpec(memory_space=pl.ANY),
        ],
        out_specs=pl.BlockSpec((tm, H), lambda c, k, tok: (c * nk + k, 0)),
        scratch_shapes=[
            pltpu.VMEM((tm, 1, H), jnp.float32),
            pltpu.VMEM((tm, 1, H), jnp.float32),
            pltpu.SemaphoreType.DMA((4,)),
            pltpu.SemaphoreType.DMA((4,)),
        ],
    )
    return pl.pallas_call(
        kernel_body,
        grid_spec=grid_spec,
        out_shape=jax.ShapeDtypeStruct((M, H), jnp.float32),
        compiler_params=pltpu.CompilerParams(
            dimension_semantics=("parallel", "arbitrary"),
            disable_bounds_checks=True),
        cost_estimate=pl.CostEstimate(
            flops=2 * M * H * H,
            transcendentals=M * H,
            bytes_accessed=(M * H * 4 + M * 4 + H * H * 2 + H * 4
                            + M * H * 4),
        ),
    )(tokens, mask_f32, w_bf16, b_f32, table)


_NT = (((1,), (1,)), ((), ()))


def _head_kernel(span_ref, seq_ref, labels_ref, pack_ref,
                 trig_ref, reps_ref, pwi_ref, cost_ref, ptj_ref,
                 buf_a0, buf_b0, buf_a1, buf_b1,
                 feat_t, pwi_t, cost_t, ptj_t,
                 sem_a0, sem_b0, sem_a1, sem_b1, osem,
                 *, n_trig, B, S, N_SPAN, C, H, TG, NSTEPS):
    k = pl.program_id(0)
    U = 8

    def issue_trig(blk, buf_a, buf_b, sem_a, sem_b):
        base = blk * TG
        for g in range(TG):
            r = base + g
            bt = r // N_SPAN
            st = r % N_SPAN
            off = bt * S
            pltpu.make_async_copy(
                seq_ref.at[pl.ds(span_ref[bt, st, 0] + off, 1)],
                buf_a.at[g], sem_a).start()
            pltpu.make_async_copy(
                seq_ref.at[pl.ds(span_ref[bt, st, 1] + off, 1)],
                buf_b.at[g], sem_b).start()

    def issue_cls(buf_a, sem_a):
        for g in range(B):
            pltpu.make_async_copy(seq_ref.at[pl.ds(g * S, 1)],
                                  buf_a.at[g], sem_a).start()

    def consume_trig(blk, buf_a, buf_b, sem_a, sem_b):
        base = blk * TG
        labels = labels_ref[...]
        l2_row = pack_ref[4:5, 0:C]
        b_trig = pack_ref[3:4, 0:1]

        pltpu.make_async_copy(buf_a.at[pl.ds(0, TG)],
                              buf_a.at[pl.ds(0, TG)], sem_a).wait()
        pltpu.make_async_copy(buf_b.at[pl.ds(0, TG)],
                              buf_b.at[pl.ds(0, TG)], sem_b).wait()

        feat = (buf_a[...].reshape(TG, H) + buf_b[...].reshape(TG, H)) * 0.5
        feat_t[...] = feat

        fdots = jax.lax.dot_general(feat, pack_ref[...], _NT,
                                    preferred_element_type=jnp.float32)
        lab_dot = jax.lax.dot_general(feat, labels, _NT,
                                      preferred_element_type=jnp.float32)
        t2 = jnp.sum(feat * feat, axis=-1, keepdims=True)

        pwi_t[...] = jax.nn.sigmoid(fdots[:, 0:1] + b_trig)
        cost_t[...] = jnp.sqrt(jnp.maximum(t2 + l2_row - 2.0 * lab_dot, 0.0))

        c1 = pltpu.make_async_copy(feat_t, trig_ref.at[pl.ds(base, TG)], osem)
        c2 = pltpu.make_async_copy(pwi_t, pwi_ref.at[pl.ds(base, TG)], osem)
        c3 = pltpu.make_async_copy(cost_t, cost_ref.at[pl.ds(base, TG)], osem)
        c1.start(); c2.start(); c3.start()
        c1.wait(); c2.wait(); c3.wait()

    def consume_cls(buf_a, sem_a):
        labw2_row = pack_ref[5:6, 0:C]
        b_type = pack_ref[3:4, 1:2]

        pltpu.make_async_copy(buf_a.at[pl.ds(0, B)],
                              buf_a.at[pl.ds(0, B)], sem_a).wait()

        feat = buf_a[pl.ds(0, B)].reshape(B, H)
        feat_t[pl.ds(0, B)] = feat

        fdots = jax.lax.dot_general(feat, pack_ref[...], _NT,
                                    preferred_element_type=jnp.float32)
        ptj_t[pl.ds(0, B)] = jax.nn.sigmoid(fdots[:, 1:2] + labw2_row
                                            + b_type)

        c1 = pltpu.make_async_copy(feat_t.at[pl.ds(0, B)], reps_ref, osem)
        c2 = pltpu.make_async_copy(ptj_t.at[pl.ds(0, B)], ptj_ref, osem)
        c1.start(); c2.start()
        c1.wait(); c2.wait()

    even = (k % 2) == 0

    @pl.when(k == 0)
    def _():
        issue_trig(k, buf_a0, buf_b0, sem_a0, sem_b0)

    @pl.when(jnp.logical_and(even, k < NSTEPS - 1))
    def _():
        issue_trig(k + 1, buf_a1, buf_b1, sem_a1, sem_b1)
        consume_trig(k, buf_a0, buf_b0, sem_a0, sem_b0)

    @pl.when(jnp.logical_and(jnp.logical_not(even), k < NSTEPS - 2))
    def _():
        issue_trig(k + 1, buf_a0, buf_b0, sem_a0, sem_b0)
        consume_trig(k, buf_a1, buf_b1, sem_a1, sem_b1)

    @pl.when(k == NSTEPS - 2)
    def _():
        issue_cls(buf_a0, sem_a0)
        consume_trig(k, buf_a1, buf_b1, sem_a1, sem_b1)

    @pl.when(k == NSTEPS - 1)
    def _():
        consume_cls(buf_a0, sem_a0)


def _head_forward(span, seq, labels, pack4, *, n_trig, B, S, C, tg):
    M, H = seq.shape
    N_SPAN = span.shape[1]
    GR_pad = n_trig + tg
    nsteps = GR_pad // tg
    kernel_body = functools.partial(
        _head_kernel, n_trig=n_trig, B=B, S=S, N_SPAN=N_SPAN, C=C, H=H,
        TG=tg, NSTEPS=nsteps)
    grid_spec = pltpu.PrefetchScalarGridSpec(
        num_scalar_prefetch=1,
        grid=(GR_pad // tg,),
        in_specs=[
            pl.BlockSpec(memory_space=pl.ANY),
            pl.BlockSpec((C, H), lambda i, s: (0, 0)),
            pl.BlockSpec((6, H), lambda i, s: (0, 0)),
        ],
        out_specs=(
            pl.BlockSpec(memory_space=pl.ANY),
            pl.BlockSpec(memory_space=pl.ANY),
            pl.BlockSpec(memory_space=pl.ANY),
            pl.BlockSpec(memory_space=pl.ANY),
            pl.BlockSpec(memory_space=pl.ANY),
        ),
        scratch_shapes=[
            pltpu.VMEM((tg, 1, H), jnp.float32),
            pltpu.VMEM((tg, 1, H), jnp.float32),
            pltpu.VMEM((tg, 1, H), jnp.float32),
            pltpu.VMEM((tg, 1, H), jnp.float32),
            pltpu.VMEM((tg, H), jnp.float32),
            pltpu.VMEM((tg, 1), jnp.float32),
            pltpu.VMEM((tg, C), jnp.float32),
            pltpu.VMEM((tg, C), jnp.float32),
            pltpu.SemaphoreType.DMA,
            pltpu.SemaphoreType.DMA,
            pltpu.SemaphoreType.DMA,
            pltpu.SemaphoreType.DMA,
            pltpu.SemaphoreType.DMA,
        ],
    )
    out_shapes = (
        jax.ShapeDtypeStruct((n_trig, H), jnp.float32),
        jax.ShapeDtypeStruct((B, H), jnp.float32),
        jax.ShapeDtypeStruct((n_trig, 1), jnp.float32),
        jax.ShapeDtypeStruct((n_trig, C), jnp.float32),
        jax.ShapeDtypeStruct((B, C), jnp.float32),
    )
    return pl.pallas_call(
        kernel_body,
        grid_spec=grid_spec,
        out_shape=out_shapes,
        compiler_params=pltpu.CompilerParams(
            dimension_semantics=("arbitrary",),
            disable_bounds_checks=True,
            vmem_limit_bytes=32 * 1024 * 1024,
        ),
        cost_estimate=pl.CostEstimate(
            flops=2 * GR_pad * H * (C + 4) + 2 * C * C * H,
            transcendentals=2 * GR_pad * C,
            bytes_accessed=(2 * GR_pad * H * 4 + C * H * 4
                            + GR_pad * (H + C + 1) * 4),
        ),
    )(span, seq, labels, pack4)


def kernel(emb_table, w_enc, b_enc, label_embeddings, w_trig, b_trig,
           w_type, b_type, x_tokens, masks, span):
    B, S = x_tokens.shape
    V, H = emb_table.shape
    C = label_embeddings.shape[0]
    N_SPAN = span.shape[1]
    M = B * S
    n_trig = B * N_SPAN

    tokens = x_tokens.reshape(-1).astype(jnp.int32)
    mask_flat = masks.reshape(-1, 1).astype(jnp.float32)
    seq_f32 = _encoder_forward(tokens, mask_flat, w_enc, b_enc, emb_table)

    labels = label_embeddings
    bias_row = jnp.pad(jnp.concatenate([b_trig, b_type], axis=1),
                       ((0, 0), (0, H - 2)))
    l2_row = jnp.pad(jnp.sum(labels * labels, axis=1)[None, :],
                     ((0, 0), (0, H - C)))
    labw2_row = jnp.pad((labels @ w_type[H:])[:, 0][None, :],
                        ((0, 0), (0, H - C)))
    pack6 = jnp.concatenate(
        [w_trig.T, w_type.reshape(2, H), bias_row, l2_row, labw2_row],
        axis=0)

    trig_feat, reps, p_wi, cost, ptj2 = _head_forward(
        span.astype(jnp.int32), seq_f32, labels, pack6,
        n_trig=n_trig, B=B, S=S, C=C, tg=128)

    p_tj = ptj2[..., None]
    return {
        "reps": reps,
        "context_feat": seq_f32,
        "trig_feat": trig_feat,
        "p_wi": p_wi,
        "D_W_P": jnp.ones_like(p_wi),
        "p_tj": p_tj,
        "D_T_P": jnp.ones_like(p_tj),
        "cost_matrix": cost,
    }
